# baseline (speedup 1.0000x reference)
import functools

import jax
import jax.numpy as jnp
from jax import lax
from jax.experimental import pallas as pl
from jax.experimental.pallas import tpu as pltpu
from jax.experimental.pallas import tpu_sc as plsc

B = 16384
D = 16
V = 100000
KP1 = 65
ROWS = KP1 * B
NW = 32
RPW = ROWS // NW
CHUNK = 128
CPG = 4
GROUP_ROWS = CPG * CHUNK
NG = RPW // GROUP_ROWS
SLAB = V // NW
NGRP_Y = B // 16
MOM = 0.5


def _rsqrt16(x):
    i = lax.bitcast_convert_type(x, jnp.int32)
    i = jnp.int32(0x5F3759DF) - lax.shift_right_logical(i, 1)
    y = lax.bitcast_convert_type(i, jnp.float32)
    for _ in range(3):
        y = y * (1.5 - 0.5 * x * y * y)
    return y


def _sc_body(idx_hbm, y_hbm, v1_hbm, v2_hbm, m1_hbm, m2_hbm, m2bf_hbm,
             g_hbm, nm1_hbm, nm2_hbm,
             idx_v0, idx_v1, rows_v0, rows_v1,
             y_v, ids_v, yvals_v, slab_v, vrows_v, mrows_v, urows_v,
             semI0, semI1, semG, semO, semC):
    cid = lax.axis_index("c")
    sid = lax.axis_index("s")
    wid = sid * 2 + cid
    base = wid * RPW
    iota16 = lax.iota(jnp.int32, 16)

    def fire_idx(grp, buf, sem):
        pltpu.async_copy(idx_hbm.at[pl.ds(base + grp * GROUP_ROWS, GROUP_ROWS)],
                         buf, sem)

    def wait_idx(buf, sem):
        pltpu.make_async_copy(idx_hbm.at[pl.ds(0, GROUP_ROWS)], buf, sem).wait()

    def drain_outs(rows_buf):
        pltpu.make_async_copy(g_hbm.at[0, pl.ds(0, GROUP_ROWS)], rows_buf,
                              semO).wait()

    def gather_group(idx_buf, rows_buf):
        hs = []
        for ci in range(CPG):
            hs.append(pltpu.async_copy(
                m2bf_hbm.at[idx_buf.at[pl.ds(ci * CHUNK, CHUNK)]],
                rows_buf.at[pl.ds(ci * CHUNK, CHUNK)], semG))
        return hs

    def out_group(grp, rows_buf):
        for ci in range(CPG):
            r0 = base + grp * GROUP_ROWS + ci * CHUNK
            pltpu.async_copy(rows_buf.at[pl.ds(ci * CHUNK, CHUNK)],
                             g_hbm.at[r0 // B, pl.ds(r0 % B, CHUNK)], semO)

    def half(t, grp, idx_buf, rows_buf, sem_this, nxt):
        wait_idx(idx_buf, sem_this)

        @pl.when(t >= 1)
        def _():
            drain_outs(rows_buf)
        hs = gather_group(idx_buf, rows_buf)
        nxt()
        for h in hs:
            h.wait()
        out_group(grp, rows_buf)

    fire_idx(0, idx_v0, semI0)

    def pair(t, carry):
        half(t, 2 * t, idx_v0, rows_v0, semI0,
             lambda: fire_idx(2 * t + 1, idx_v1, semI1))
        half(t, 2 * t + 1, idx_v1, rows_v1, semI1,
             lambda: fire_idx(2 * t + 2, idx_v0, semI0))
        return carry

    lax.fori_loop(0, NG // 2, pair, 0)
    wait_idx(idx_v0, semI0)
    drain_outs(rows_v0)
    hs = gather_group(idx_v0, rows_v0)
    for h in hs:
        h.wait()
    drain_outs(rows_v1)
    out_group(NG - 1, rows_v0)
    drain_outs(rows_v0)

    lo = wid * SLAB

    pltpu.sync_copy(y_hbm, y_v)

    def memset_body(j, c):
        z = jnp.zeros((16,), jnp.int32)
        ids_v[pl.ds(j * 16, 16)] = z
        yvals_v[pl.ds(j * 16, 16)] = z
        return c

    lax.fori_loop(0, (B + 128) // 16, memset_body, 0)

    def scan_body(j, count):
        yv = y_v[pl.ds(j * 16, 16)]
        lov = jnp.full((16,), lo, jnp.int32)
        hiv = jnp.full((16,), lo + SLAB, jnp.int32)
        m = (yv >= lov) & (yv < hiv)
        mi = jnp.where(m, jnp.ones((16,), jnp.int32), jnp.zeros((16,), jnp.int32))
        pos = jnp.full((16,), count - 1, jnp.int32) + plsc.cumsum(mi)
        plsc.store_scatter(ids_v, [pos],
                           jnp.full((16,), j * 16, jnp.int32) + iota16, mask=m)
        plsc.store_scatter(yvals_v, [pos], yv, mask=m)
        return count + jnp.sum(mi)

    count = lax.fori_loop(0, NGRP_Y, scan_body, 0)
    nchunks = (count + CHUNK - 1) // CHUNK

    def do_bank(mem_hbm, v_hbm, nm_hbm):
        pltpu.sync_copy(mem_hbm.at[pl.ds(lo, SLAB)], slab_v)

        def chunk_body(c, carry):
            c0 = c * CHUNK
            pltpu.async_copy(v_hbm.at[ids_v.at[pl.ds(c0, CHUNK)]],
                             vrows_v, semC).wait()
            pltpu.async_copy(mem_hbm.at[yvals_v.at[pl.ds(c0, CHUNK)]],
                             mrows_v, semC).wait()

            def sub_body(s, c2):
                rowid = s * 16 + iota16
                comps = []
                acc = jnp.zeros((16,), jnp.float32)
                for jc in range(D):
                    jv = jnp.full((16,), jc, jnp.int32)
                    vj = plsc.load_gather(vrows_v, [rowid, jv])
                    mj = plsc.load_gather(mrows_v, [rowid, jv])
                    bj = MOM * mj + (1.0 - MOM) * vj
                    comps.append(bj)
                    acc = acc + bj * bj
                rinv = _rsqrt16(acc)
                for jc in range(D):
                    jv = jnp.full((16,), jc, jnp.int32)
                    plsc.store_scatter(urows_v, [rowid, jv], comps[jc] * rinv)
                return c2

            lax.fori_loop(0, CHUNK // 16, sub_body, 0)
            rem = jnp.minimum(CHUNK, count - c0)

            def st_body(q, c2):
                yq = yvals_v[pl.ds(c0 + q, 16)][0]
                locv = jnp.full((16,), yq - lo, jnp.int32)
                qv = jnp.full((16,), q, jnp.int32)
                row = plsc.load_gather(urows_v, [qv, iota16])
                plsc.store_scatter(slab_v, [locv, iota16], row)
                return c2

            lax.fori_loop(0, rem, st_body, 0)
            return carry

        lax.fori_loop(0, nchunks, chunk_body, 0)
        pltpu.sync_copy(slab_v, nm_hbm.at[pl.ds(lo, SLAB)])

    do_bank(m1_hbm, v1_hbm, nm1_hbm)
    do_bank(m2_hbm, v2_hbm, nm2_hbm)


_sc_call = functools.partial(
    pl.kernel,
    out_type=[
        jax.ShapeDtypeStruct((KP1, B, D), jnp.bfloat16),
        jax.ShapeDtypeStruct((V, D), jnp.float32),
        jax.ShapeDtypeStruct((V, D), jnp.float32),
    ],
    mesh=plsc.VectorSubcoreMesh(core_axis_name="c", subcore_axis_name="s"),
    compiler_params=pltpu.CompilerParams(use_tc_tiling_on_sc=False,
                                         needs_layout_passes=False),
    scratch_types=[
        pltpu.VMEM((GROUP_ROWS,), jnp.int32),
        pltpu.VMEM((GROUP_ROWS,), jnp.int32),
        pltpu.VMEM((GROUP_ROWS, D), jnp.bfloat16),
        pltpu.VMEM((GROUP_ROWS, D), jnp.bfloat16),
        pltpu.VMEM((B,), jnp.int32),
        pltpu.VMEM((B + 128,), jnp.int32),
        pltpu.VMEM((B + 128,), jnp.int32),
        pltpu.VMEM((SLAB, D), jnp.float32),
        pltpu.VMEM((CHUNK, D), jnp.float32),
        pltpu.VMEM((CHUNK, D), jnp.float32),
        pltpu.VMEM((CHUNK, D), jnp.float32),
        pltpu.SemaphoreType.DMA,
        pltpu.SemaphoreType.DMA,
        pltpu.SemaphoreType.DMA,
        pltpu.SemaphoreType.DMA,
        pltpu.SemaphoreType.DMA,
    ],
)(_sc_body)


BT = 4096


def _tc_body(g_ref, v1_ref, v2_ref,
             Ws2_ref, Wt2_ref, Ws1_ref, Wt1_ref, Wsv_ref, Wtv_ref,
             cs_ref, ct_ref, bsv_ref, btv_ref,
             outt_ref, outs_ref, s1T, t1T):
    k = pl.program_id(1)

    def dotT(W, x):
        return lax.dot_general(W, x, (((1,), (0,)), ((), ())),
                               precision=lax.Precision.DEFAULT,
                               preferred_element_type=jnp.float32)

    @pl.when(k == 0)
    def _():
        s1T[...] = dotT(Ws1_ref[...], v1_ref[...]) + cs_ref[...]
        t1T[...] = dotT(Wt1_ref[...], v2_ref[...]) + ct_ref[...]

    g = g_ref[0]
    bf = jnp.bfloat16
    rs = jnp.maximum(s1T[...] - dotT(Ws2_ref[...].astype(bf), g), 0.0)
    outs_ref[0] = dotT(Wsv_ref[...], rs) + bsv_ref[...]
    rt = jnp.maximum(t1T[...] - dotT(Wt2_ref[...].astype(bf), g), 0.0)
    outt_ref[0] = dotT(Wtv_ref[...], rt) + btv_ref[...]


def _tc_call(gT, v1T, v2T, Ws2, Wt2, Ws1, Wt1, Wsv, Wtv, cs, ct, bsv, btv):
    wspec = pl.BlockSpec((D, D), lambda i, k: (0, 0))
    bspec = pl.BlockSpec((D, 1), lambda i, k: (0, 0))
    return pl.pallas_call(
        _tc_body,
        grid=(B // BT, KP1),
        in_specs=[
            pl.BlockSpec((1, D, BT), lambda i, k: (k, 0, i)),
            pl.BlockSpec((D, BT), lambda i, k: (0, i)),
            pl.BlockSpec((D, BT), lambda i, k: (0, i)),
            wspec, wspec, wspec, wspec, wspec, wspec,
            bspec, bspec, bspec, bspec,
        ],
        out_specs=[
            pl.BlockSpec((1, D, BT), lambda i, k: (k, 0, i)),
            pl.BlockSpec((1, D, BT), lambda i, k: (k, 0, i)),
        ],
        out_shape=[
            jax.ShapeDtypeStruct((KP1, D, B), jnp.float32),
            jax.ShapeDtypeStruct((KP1, D, B), jnp.float32),
        ],
        scratch_shapes=[
            pltpu.VMEM((D, BT), jnp.float32),
            pltpu.VMEM((D, BT), jnp.float32),
        ],
    )(gT, v1T, v2T, Ws2, Wt2, Ws1, Wt1, Wsv, Wtv, cs, ct, bsv, btv)


def kernel(v1, v2, y, idx, memory_v1, memory_v2,
           w_s_v1_W, w_s_v1_b, w_s_v2_W, w_s_v2_b, w_s_v_W, w_s_v_b,
           w_t_v1_W, w_t_v1_b, w_t_v2_W, w_t_v2_b, w_t_v_W, w_t_v_b):
    idx_t = idx.T.reshape(-1)
    m2_bf = memory_v2.astype(jnp.bfloat16)
    g3, nm1, nm2 = _sc_call(idx_t, y, v1, v2, memory_v1, memory_v2, m2_bf)
    gT = g3.transpose(0, 2, 1)
    cs = (w_s_v1_b - w_s_v2_b).reshape(D, 1)
    ct = (w_t_v1_b - w_t_v2_b).reshape(D, 1)
    outtT, outsT = _tc_call(gT, v1.T, v2.T, w_s_v2_W, w_t_v2_W, w_s_v1_W,
                            w_t_v1_W, w_s_v_W, w_t_v_W, cs, ct,
                            w_s_v_b.reshape(D, 1), w_t_v_b.reshape(D, 1))
    return (outtT.transpose(0, 2, 1), outsT.transpose(0, 2, 1), nm1, nm2)


# speedup vs baseline: 1.2053x; 1.2053x over previous
import functools

import jax
import jax.numpy as jnp
from jax import lax
from jax.experimental import pallas as pl
from jax.experimental.pallas import tpu as pltpu
from jax.experimental.pallas import tpu_sc as plsc

B = 16384
D = 16
V = 100000
KP1 = 65
ROWS = KP1 * B
NW = 32
RPW = ROWS // NW
CHUNK = 128
CPG = 4
GROUP_ROWS = CPG * CHUNK
NG = RPW // GROUP_ROWS
SLAB = V // NW
NGRP_Y = B // 16
MOM = 0.5


def _rsqrt16(x):
    i = lax.bitcast_convert_type(x, jnp.int32)
    i = jnp.int32(0x5F3759DF) - lax.shift_right_logical(i, 1)
    y = lax.bitcast_convert_type(i, jnp.float32)
    for _ in range(3):
        y = y * (1.5 - 0.5 * x * y * y)
    return y


def _sc_gather_body(idx_hbm, m2_hbm, g_hbm,
                    idx_v0, idx_v1, rows_v0, rows_v1,
                    semI0, semI1, semG, semO):
    cid = lax.axis_index("c")
    sid = lax.axis_index("s")
    wid = sid * 2 + cid
    base = wid * RPW

    def fire_idx(grp, buf, sem):
        pltpu.async_copy(idx_hbm.at[pl.ds(base + grp * GROUP_ROWS, GROUP_ROWS)],
                         buf, sem)

    def wait_idx(buf, sem):
        pltpu.make_async_copy(idx_hbm.at[pl.ds(0, GROUP_ROWS)], buf, sem).wait()

    def drain_outs(rows_buf):
        pltpu.make_async_copy(g_hbm.at[0, pl.ds(0, GROUP_ROWS)], rows_buf,
                              semO).wait()

    def gather_group(idx_buf, rows_buf):
        hs = []
        for ci in range(CPG):
            hs.append(pltpu.async_copy(
                m2_hbm.at[idx_buf.at[pl.ds(ci * CHUNK, CHUNK)]],
                rows_buf.at[pl.ds(ci * CHUNK, CHUNK)], semG))
        return hs

    def out_group(grp, rows_buf):
        for ci in range(CPG):
            r0 = base + grp * GROUP_ROWS + ci * CHUNK
            pltpu.async_copy(rows_buf.at[pl.ds(ci * CHUNK, CHUNK)],
                             g_hbm.at[r0 // B, pl.ds(r0 % B, CHUNK)], semO)

    def half(t, grp, idx_buf, rows_buf, sem_this, nxt):
        wait_idx(idx_buf, sem_this)

        @pl.when(t >= 1)
        def _():
            drain_outs(rows_buf)
        hs = gather_group(idx_buf, rows_buf)
        nxt()
        for h in hs:
            h.wait()
        out_group(grp, rows_buf)

    fire_idx(0, idx_v0, semI0)

    def pair(t, carry):
        half(t, 2 * t, idx_v0, rows_v0, semI0,
             lambda: fire_idx(2 * t + 1, idx_v1, semI1))
        half(t, 2 * t + 1, idx_v1, rows_v1, semI1,
             lambda: fire_idx(2 * t + 2, idx_v0, semI0))
        return carry

    lax.fori_loop(0, NG // 2, pair, 0)
    wait_idx(idx_v0, semI0)
    drain_outs(rows_v0)
    hs = gather_group(idx_v0, rows_v0)
    for h in hs:
        h.wait()
    drain_outs(rows_v1)
    out_group(NG - 1, rows_v0)
    drain_outs(rows_v0)


_sc_gather = functools.partial(
    pl.kernel,
    out_type=jax.ShapeDtypeStruct((KP1, B, D), jnp.float32),
    mesh=plsc.VectorSubcoreMesh(core_axis_name="c", subcore_axis_name="s"),
    compiler_params=pltpu.CompilerParams(use_tc_tiling_on_sc=False,
                                         needs_layout_passes=False),
    scratch_types=[
        pltpu.VMEM((GROUP_ROWS,), jnp.int32),
        pltpu.VMEM((GROUP_ROWS,), jnp.int32),
        pltpu.VMEM((GROUP_ROWS, D), jnp.float32),
        pltpu.VMEM((GROUP_ROWS, D), jnp.float32),
        pltpu.SemaphoreType.DMA,
        pltpu.SemaphoreType.DMA,
        pltpu.SemaphoreType.DMA,
        pltpu.SemaphoreType.DMA,
    ],
)(_sc_gather_body)


def _sc_update_body(y_hbm, v1_hbm, v2_hbm, m1_hbm, m2_hbm,
                    nm1_hbm, nm2_hbm,
                    y_v, ids_v, yvals_v, slab_v, vrows_v, mrows_v, urows_v,
                    semC):
    cid = lax.axis_index("c")
    sid = lax.axis_index("s")
    wid = sid * 2 + cid
    iota16 = lax.iota(jnp.int32, 16)
    lo = wid * SLAB

    pltpu.sync_copy(y_hbm, y_v)

    def memset_body(j, c):
        z = jnp.zeros((16,), jnp.int32)
        ids_v[pl.ds(j * 16, 16)] = z
        yvals_v[pl.ds(j * 16, 16)] = z
        return c

    lax.fori_loop(0, (B + 128) // 16, memset_body, 0)

    def scan_body(j, count):
        yv = y_v[pl.ds(j * 16, 16)]
        lov = jnp.full((16,), lo, jnp.int32)
        hiv = jnp.full((16,), lo + SLAB, jnp.int32)
        m = (yv >= lov) & (yv < hiv)
        mi = jnp.where(m, jnp.ones((16,), jnp.int32), jnp.zeros((16,), jnp.int32))
        pos = jnp.full((16,), count - 1, jnp.int32) + plsc.cumsum(mi)
        plsc.store_scatter(ids_v, [pos],
                           jnp.full((16,), j * 16, jnp.int32) + iota16, mask=m)
        plsc.store_scatter(yvals_v, [pos], yv, mask=m)
        return count + jnp.sum(mi)

    count = lax.fori_loop(0, NGRP_Y, scan_body, 0)
    nchunks = (count + CHUNK - 1) // CHUNK

    def do_bank(mem_hbm, v_hbm, nm_hbm):
        pltpu.sync_copy(mem_hbm.at[pl.ds(lo, SLAB)], slab_v)

        def chunk_body(c, carry):
            c0 = c * CHUNK
            pltpu.async_copy(v_hbm.at[ids_v.at[pl.ds(c0, CHUNK)]],
                             vrows_v, semC).wait()
            pltpu.async_copy(mem_hbm.at[yvals_v.at[pl.ds(c0, CHUNK)]],
                             mrows_v, semC).wait()

            def sub_body(s, c2):
                rowid = s * 16 + iota16
                comps = []
                acc = jnp.zeros((16,), jnp.float32)
                for jc in range(D):
                    jv = jnp.full((16,), jc, jnp.int32)
                    vj = plsc.load_gather(vrows_v, [rowid, jv])
                    mj = plsc.load_gather(mrows_v, [rowid, jv])
                    bj = MOM * mj + (1.0 - MOM) * vj
                    comps.append(bj)
                    acc = acc + bj * bj
                rinv = _rsqrt16(acc)
                for jc in range(D):
                    jv = jnp.full((16,), jc, jnp.int32)
                    plsc.store_scatter(urows_v, [rowid, jv], comps[jc] * rinv)
                return c2

            lax.fori_loop(0, CHUNK // 16, sub_body, 0)
            rem = jnp.minimum(CHUNK, count - c0)

            def st_body(q, c2):
                yq = yvals_v[pl.ds(c0 + q, 16)][0]
                locv = jnp.full((16,), yq - lo, jnp.int32)
                qv = jnp.full((16,), q, jnp.int32)
                row = plsc.load_gather(urows_v, [qv, iota16])
                plsc.store_scatter(slab_v, [locv, iota16], row)
                return c2

            lax.fori_loop(0, rem, st_body, 0)
            return carry

        lax.fori_loop(0, nchunks, chunk_body, 0)
        pltpu.sync_copy(slab_v, nm_hbm.at[pl.ds(lo, SLAB)])

    do_bank(m1_hbm, v1_hbm, nm1_hbm)
    do_bank(m2_hbm, v2_hbm, nm2_hbm)


_sc_update = functools.partial(
    pl.kernel,
    out_type=[
        jax.ShapeDtypeStruct((V, D), jnp.float32),
        jax.ShapeDtypeStruct((V, D), jnp.float32),
    ],
    mesh=plsc.VectorSubcoreMesh(core_axis_name="c", subcore_axis_name="s"),
    compiler_params=pltpu.CompilerParams(use_tc_tiling_on_sc=False,
                                         needs_layout_passes=False),
    scratch_types=[
        pltpu.VMEM((B,), jnp.int32),
        pltpu.VMEM((B + 128,), jnp.int32),
        pltpu.VMEM((B + 128,), jnp.int32),
        pltpu.VMEM((SLAB, D), jnp.float32),
        pltpu.VMEM((CHUNK, D), jnp.float32),
        pltpu.VMEM((CHUNK, D), jnp.float32),
        pltpu.VMEM((CHUNK, D), jnp.float32),
        pltpu.SemaphoreType.DMA,
    ],
)(_sc_update_body)


BT = 4096


def _tc_body(g_ref, v1_ref, v2_ref,
             Ws2_ref, Wt2_ref, Ws1_ref, Wt1_ref, Wsv_ref, Wtv_ref,
             cs_ref, ct_ref, bsv_ref, btv_ref,
             outt_ref, outs_ref, s1T, t1T):
    k = pl.program_id(1)

    def dotT(W, x):
        return lax.dot_general(W, x, (((1,), (0,)), ((), ())),
                               precision=lax.Precision.DEFAULT,
                               preferred_element_type=jnp.float32)

    @pl.when(k == 0)
    def _():
        s1T[...] = dotT(Ws1_ref[...], v1_ref[...]) + cs_ref[...]
        t1T[...] = dotT(Wt1_ref[...], v2_ref[...]) + ct_ref[...]

    g = g_ref[0]
    rs = jnp.maximum(s1T[...] - dotT(Ws2_ref[...], g), 0.0)
    outs_ref[0] = dotT(Wsv_ref[...], rs) + bsv_ref[...]
    rt = jnp.maximum(t1T[...] - dotT(Wt2_ref[...], g), 0.0)
    outt_ref[0] = dotT(Wtv_ref[...], rt) + btv_ref[...]


def _tc_call(gT, v1T, v2T, Ws2, Wt2, Ws1, Wt1, Wsv, Wtv, cs, ct, bsv, btv):
    wspec = pl.BlockSpec((D, D), lambda i, k: (0, 0))
    bspec = pl.BlockSpec((D, 1), lambda i, k: (0, 0))
    return pl.pallas_call(
        _tc_body,
        grid=(B // BT, KP1),
        in_specs=[
            pl.BlockSpec((1, D, BT), lambda i, k: (k, 0, i)),
            pl.BlockSpec((D, BT), lambda i, k: (0, i)),
            pl.BlockSpec((D, BT), lambda i, k: (0, i)),
            wspec, wspec, wspec, wspec, wspec, wspec,
            bspec, bspec, bspec, bspec,
        ],
        out_specs=[
            pl.BlockSpec((1, D, BT), lambda i, k: (k, 0, i)),
            pl.BlockSpec((1, D, BT), lambda i, k: (k, 0, i)),
        ],
        out_shape=[
            jax.ShapeDtypeStruct((KP1, D, B), jnp.float32),
            jax.ShapeDtypeStruct((KP1, D, B), jnp.float32),
        ],
        scratch_shapes=[
            pltpu.VMEM((D, BT), jnp.float32),
            pltpu.VMEM((D, BT), jnp.float32),
        ],
    )(gT, v1T, v2T, Ws2, Wt2, Ws1, Wt1, Wsv, Wtv, cs, ct, bsv, btv)


def kernel(v1, v2, y, idx, memory_v1, memory_v2,
           w_s_v1_W, w_s_v1_b, w_s_v2_W, w_s_v2_b, w_s_v_W, w_s_v_b,
           w_t_v1_W, w_t_v1_b, w_t_v2_W, w_t_v2_b, w_t_v_W, w_t_v_b):
    idx_t = idx.T.reshape(-1)
    g3 = _sc_gather(idx_t, memory_v2)
    nm1, nm2 = _sc_update(y, v1, v2, memory_v1, memory_v2)
    gT = g3.transpose(0, 2, 1)
    cs = (w_s_v1_b - w_s_v2_b).reshape(D, 1)
    ct = (w_t_v1_b - w_t_v2_b).reshape(D, 1)
    outtT, outsT = _tc_call(gT, v1.T, v2.T, w_s_v2_W, w_t_v2_W, w_s_v1_W,
                            w_t_v1_W, w_s_v_W, w_t_v_W, cs, ct,
                            w_s_v_b.reshape(D, 1), w_t_v_b.reshape(D, 1))
    return (outtT.transpose(0, 2, 1), outsT.transpose(0, 2, 1), nm1, nm2)


# speedup vs baseline: 1.3252x; 1.0994x over previous
import functools

import jax
import jax.numpy as jnp
from jax import lax
from jax.experimental import pallas as pl
from jax.experimental.pallas import tpu as pltpu
from jax.experimental.pallas import tpu_sc as plsc

B = 16384
D = 16
V = 100000
KP1 = 65
ROWS = KP1 * B
NW = 32
RPW = ROWS // NW
CHUNK = 128
CPG = 4
GROUP_ROWS = CPG * CHUNK
NG = RPW // GROUP_ROWS
SLAB = V // NW
NGRP_Y = B // 16
MOM = 0.5


def _rsqrt16(x):
    i = lax.bitcast_convert_type(x, jnp.int32)
    i = jnp.int32(0x5F3759DF) - lax.shift_right_logical(i, 1)
    y = lax.bitcast_convert_type(i, jnp.float32)
    for _ in range(3):
        y = y * (1.5 - 0.5 * x * y * y)
    return y


def _sc_gather_body(idx_hbm, m2_hbm, g_hbm,
                    idx_v0, idx_v1, rows_v0, rows_v1,
                    semI0, semI1, semG, semO):
    cid = lax.axis_index("c")
    sid = lax.axis_index("s")
    wid = sid * 2 + cid
    base = wid * RPW

    def fire_idx(grp, buf, sem):
        pltpu.async_copy(idx_hbm.at[pl.ds(base + grp * GROUP_ROWS, GROUP_ROWS)],
                         buf, sem)

    def wait_idx(buf, sem):
        pltpu.make_async_copy(idx_hbm.at[pl.ds(0, GROUP_ROWS)], buf, sem).wait()

    def drain_outs(rows_buf):
        pltpu.make_async_copy(g_hbm.at[0, pl.ds(0, GROUP_ROWS)], rows_buf,
                              semO).wait()

    def gather_group(idx_buf, rows_buf):
        hs = []
        for ci in range(CPG):
            hs.append(pltpu.async_copy(
                m2_hbm.at[idx_buf.at[pl.ds(ci * CHUNK, CHUNK)]],
                rows_buf.at[pl.ds(ci * CHUNK, CHUNK)], semG))
        return hs

    def out_group(grp, rows_buf):
        for ci in range(CPG):
            r0 = base + grp * GROUP_ROWS + ci * CHUNK
            pltpu.async_copy(rows_buf.at[pl.ds(ci * CHUNK, CHUNK)],
                             g_hbm.at[r0 // B, pl.ds(r0 % B, CHUNK)], semO)

    def half(t, grp, idx_buf, rows_buf, sem_this, nxt):
        wait_idx(idx_buf, sem_this)

        @pl.when(t >= 1)
        def _():
            drain_outs(rows_buf)
        hs = gather_group(idx_buf, rows_buf)
        nxt()
        for h in hs:
            h.wait()
        out_group(grp, rows_buf)

    fire_idx(0, idx_v0, semI0)

    def pair(t, carry):
        half(t, 2 * t, idx_v0, rows_v0, semI0,
             lambda: fire_idx(2 * t + 1, idx_v1, semI1))
        half(t, 2 * t + 1, idx_v1, rows_v1, semI1,
             lambda: fire_idx(2 * t + 2, idx_v0, semI0))
        return carry

    lax.fori_loop(0, NG // 2, pair, 0)
    wait_idx(idx_v0, semI0)
    drain_outs(rows_v0)
    hs = gather_group(idx_v0, rows_v0)
    for h in hs:
        h.wait()
    drain_outs(rows_v1)
    out_group(NG - 1, rows_v0)
    drain_outs(rows_v0)


_sc_gather = functools.partial(
    pl.kernel,
    out_type=jax.ShapeDtypeStruct((KP1, B, D), jnp.float32),
    mesh=plsc.VectorSubcoreMesh(core_axis_name="c", subcore_axis_name="s"),
    compiler_params=pltpu.CompilerParams(use_tc_tiling_on_sc=False,
                                         needs_layout_passes=False),
    scratch_types=[
        pltpu.VMEM((GROUP_ROWS,), jnp.int32),
        pltpu.VMEM((GROUP_ROWS,), jnp.int32),
        pltpu.VMEM((GROUP_ROWS, D), jnp.float32),
        pltpu.VMEM((GROUP_ROWS, D), jnp.float32),
        pltpu.SemaphoreType.DMA,
        pltpu.SemaphoreType.DMA,
        pltpu.SemaphoreType.DMA,
        pltpu.SemaphoreType.DMA,
    ],
)(_sc_gather_body)


def _sc_update_body(y_hbm, v1_hbm, v2_hbm, m1_hbm, m2_hbm,
                    nm1_hbm, nm2_hbm,
                    y_v, ids_v, yvals_v, slab_v, vrows_v, mrows_v, urows_v,
                    semC):
    cid = lax.axis_index("c")
    sid = lax.axis_index("s")
    wid = sid * 2 + cid
    iota16 = lax.iota(jnp.int32, 16)
    lo = wid * SLAB

    pltpu.sync_copy(y_hbm, y_v)

    def memset_body(j, c):
        z = jnp.zeros((16,), jnp.int32)
        ids_v[pl.ds(j * 16, 16)] = z
        yvals_v[pl.ds(j * 16, 16)] = z
        return c

    lax.fori_loop(0, (B + 128) // 16, memset_body, 0)

    def scan_body(j, count):
        yv = y_v[pl.ds(j * 16, 16)]
        lov = jnp.full((16,), lo, jnp.int32)
        hiv = jnp.full((16,), lo + SLAB, jnp.int32)
        m = (yv >= lov) & (yv < hiv)
        mi = jnp.where(m, jnp.ones((16,), jnp.int32), jnp.zeros((16,), jnp.int32))
        pos = jnp.full((16,), count - 1, jnp.int32) + plsc.cumsum(mi)
        plsc.store_scatter(ids_v, [pos],
                           jnp.full((16,), j * 16, jnp.int32) + iota16, mask=m)
        plsc.store_scatter(yvals_v, [pos], yv, mask=m)
        return count + jnp.sum(mi)

    count = lax.fori_loop(0, NGRP_Y, scan_body, 0)
    nchunks = (count + CHUNK - 1) // CHUNK

    def do_bank(mem_hbm, v_hbm, nm_hbm):
        pltpu.sync_copy(mem_hbm.at[pl.ds(lo, SLAB)], slab_v)

        def chunk_body(c, carry):
            c0 = c * CHUNK
            pltpu.async_copy(v_hbm.at[ids_v.at[pl.ds(c0, CHUNK)]],
                             vrows_v, semC).wait()
            pltpu.async_copy(mem_hbm.at[yvals_v.at[pl.ds(c0, CHUNK)]],
                             mrows_v, semC).wait()

            def sub_body(s, c2):
                rowid = s * 16 + iota16
                comps = []
                acc = jnp.zeros((16,), jnp.float32)
                for jc in range(D):
                    jv = jnp.full((16,), jc, jnp.int32)
                    vj = plsc.load_gather(vrows_v, [rowid, jv])
                    mj = plsc.load_gather(mrows_v, [rowid, jv])
                    bj = MOM * mj + (1.0 - MOM) * vj
                    comps.append(bj)
                    acc = acc + bj * bj
                rinv = _rsqrt16(acc)
                for jc in range(D):
                    jv = jnp.full((16,), jc, jnp.int32)
                    plsc.store_scatter(urows_v, [rowid, jv], comps[jc] * rinv)
                return c2

            lax.fori_loop(0, CHUNK // 16, sub_body, 0)
            rem = jnp.minimum(CHUNK, count - c0)

            def st_body(q, c2):
                yq = yvals_v[pl.ds(c0 + q, 16)][0]
                locv = jnp.full((16,), yq - lo, jnp.int32)
                qv = jnp.full((16,), q, jnp.int32)
                row = plsc.load_gather(urows_v, [qv, iota16])
                plsc.store_scatter(slab_v, [locv, iota16], row)
                return c2

            lax.fori_loop(0, rem, st_body, 0)
            return carry

        lax.fori_loop(0, nchunks, chunk_body, 0)
        pltpu.sync_copy(slab_v, nm_hbm.at[pl.ds(lo, SLAB)])

    do_bank(m1_hbm, v1_hbm, nm1_hbm)
    do_bank(m2_hbm, v2_hbm, nm2_hbm)


_sc_update = functools.partial(
    pl.kernel,
    out_type=[
        jax.ShapeDtypeStruct((V, D), jnp.float32),
        jax.ShapeDtypeStruct((V, D), jnp.float32),
    ],
    mesh=plsc.VectorSubcoreMesh(core_axis_name="c", subcore_axis_name="s"),
    compiler_params=pltpu.CompilerParams(use_tc_tiling_on_sc=False,
                                         needs_layout_passes=False),
    scratch_types=[
        pltpu.VMEM((B,), jnp.int32),
        pltpu.VMEM((B + 128,), jnp.int32),
        pltpu.VMEM((B + 128,), jnp.int32),
        pltpu.VMEM((SLAB, D), jnp.float32),
        pltpu.VMEM((CHUNK, D), jnp.float32),
        pltpu.VMEM((CHUNK, D), jnp.float32),
        pltpu.VMEM((CHUNK, D), jnp.float32),
        pltpu.SemaphoreType.DMA,
    ],
)(_sc_update_body)


BT = 8192


def _tc_body(g_ref, v1_ref, v2_ref,
             Ws2_ref, Wt2_ref, Ws1_ref, Wt1_ref, Wsv_ref, Wtv_ref,
             cs_ref, ct_ref, bsv_ref, btv_ref,
             outt_ref, outs_ref, s1T, t1T):
    k = pl.program_id(1)

    def dotT(W, x):
        return lax.dot_general(W, x, (((1,), (0,)), ((), ())),
                               precision=lax.Precision.DEFAULT,
                               preferred_element_type=jnp.float32)

    @pl.when(k == 0)
    def _():
        s1T[...] = dotT(Ws1_ref[...], v1_ref[...]) + cs_ref[...]
        t1T[...] = dotT(Wt1_ref[...], v2_ref[...]) + ct_ref[...]

    g = g_ref[0]
    rs = jnp.maximum(s1T[...] - dotT(Ws2_ref[...], g), 0.0)
    outs_ref[0] = dotT(Wsv_ref[...], rs) + bsv_ref[...]
    rt = jnp.maximum(t1T[...] - dotT(Wt2_ref[...], g), 0.0)
    outt_ref[0] = dotT(Wtv_ref[...], rt) + btv_ref[...]


def _tc_call(gT, v1T, v2T, Ws2, Wt2, Ws1, Wt1, Wsv, Wtv, cs, ct, bsv, btv):
    wspec = pl.BlockSpec((D, D), lambda i, k: (0, 0))
    bspec = pl.BlockSpec((D, 1), lambda i, k: (0, 0))
    return pl.pallas_call(
        _tc_body,
        grid=(B // BT, KP1),
        in_specs=[
            pl.BlockSpec((1, D, BT), lambda i, k: (k, 0, i)),
            pl.BlockSpec((D, BT), lambda i, k: (0, i)),
            pl.BlockSpec((D, BT), lambda i, k: (0, i)),
            wspec, wspec, wspec, wspec, wspec, wspec,
            bspec, bspec, bspec, bspec,
        ],
        out_specs=[
            pl.BlockSpec((1, D, BT), lambda i, k: (k, 0, i)),
            pl.BlockSpec((1, D, BT), lambda i, k: (k, 0, i)),
        ],
        out_shape=[
            jax.ShapeDtypeStruct((KP1, D, B), jnp.float32),
            jax.ShapeDtypeStruct((KP1, D, B), jnp.float32),
        ],
        scratch_shapes=[
            pltpu.VMEM((D, BT), jnp.float32),
            pltpu.VMEM((D, BT), jnp.float32),
        ],
    )(gT, v1T, v2T, Ws2, Wt2, Ws1, Wt1, Wsv, Wtv, cs, ct, bsv, btv)


def kernel(v1, v2, y, idx, memory_v1, memory_v2,
           w_s_v1_W, w_s_v1_b, w_s_v2_W, w_s_v2_b, w_s_v_W, w_s_v_b,
           w_t_v1_W, w_t_v1_b, w_t_v2_W, w_t_v2_b, w_t_v_W, w_t_v_b):
    idx_t = idx.T.reshape(-1)
    g3 = _sc_gather(idx_t, memory_v2)
    nm1, nm2 = _sc_update(y, v1, v2, memory_v1, memory_v2)
    gT = g3.transpose(0, 2, 1)
    cs = (w_s_v1_b - w_s_v2_b).reshape(D, 1)
    ct = (w_t_v1_b - w_t_v2_b).reshape(D, 1)
    outtT, outsT = _tc_call(gT, v1.T, v2.T, w_s_v2_W, w_t_v2_W, w_s_v1_W,
                            w_t_v1_W, w_s_v_W, w_t_v_W, cs, ct,
                            w_s_v_b.reshape(D, 1), w_t_v_b.reshape(D, 1))
    return (outtT.transpose(0, 2, 1), outsT.transpose(0, 2, 1), nm1, nm2)


# speedup vs baseline: 1.4081x; 1.0626x over previous
import functools

import jax
import jax.numpy as jnp
from jax import lax
from jax.experimental import pallas as pl
from jax.experimental.pallas import tpu as pltpu
from jax.experimental.pallas import tpu_sc as plsc

B = 16384
D = 16
V = 100000
KP1 = 65
ROWS = KP1 * B
NW = 32
RPW = ROWS // NW
CHUNK = 128
CPG = 4
GROUP_ROWS = CPG * CHUNK
NG = RPW // GROUP_ROWS
SLAB = V // NW
NGRP_Y = B // 16
MOM = 0.5


def _rsqrt16(x):
    i = lax.bitcast_convert_type(x, jnp.int32)
    i = jnp.int32(0x5F3759DF) - lax.shift_right_logical(i, 1)
    y = lax.bitcast_convert_type(i, jnp.float32)
    for _ in range(3):
        y = y * (1.5 - 0.5 * x * y * y)
    return y


def _sc_gather_body(idx_hbm, m2_hbm, g_hbm,
                    idx_v0, idx_v1, rows_v0, rows_v1,
                    semI0, semI1, semG, semO):
    cid = lax.axis_index("c")
    sid = lax.axis_index("s")
    wid = sid * 2 + cid
    base = wid * RPW

    def fire_idx(grp, buf, sem):
        pltpu.async_copy(idx_hbm.at[pl.ds(base + grp * GROUP_ROWS, GROUP_ROWS)],
                         buf, sem)

    def wait_idx(buf, sem):
        pltpu.make_async_copy(idx_hbm.at[pl.ds(0, GROUP_ROWS)], buf, sem).wait()

    def drain_outs(rows_buf):
        pltpu.make_async_copy(g_hbm.at[0, pl.ds(0, GROUP_ROWS)], rows_buf,
                              semO).wait()

    def gather_group(idx_buf, rows_buf):
        hs = []
        for ci in range(CPG):
            hs.append(pltpu.async_copy(
                m2_hbm.at[idx_buf.at[pl.ds(ci * CHUNK, CHUNK)]],
                rows_buf.at[pl.ds(ci * CHUNK, CHUNK)], semG))
        return hs

    def out_group(grp, rows_buf):
        for ci in range(CPG):
            r0 = base + grp * GROUP_ROWS + ci * CHUNK
            pltpu.async_copy(rows_buf.at[pl.ds(ci * CHUNK, CHUNK)],
                             g_hbm.at[r0 // B, pl.ds(r0 % B, CHUNK)], semO)

    def half(t, grp, idx_buf, rows_buf, sem_this, nxt):
        wait_idx(idx_buf, sem_this)

        @pl.when(t >= 1)
        def _():
            drain_outs(rows_buf)
        hs = gather_group(idx_buf, rows_buf)
        nxt()
        for h in hs:
            h.wait()
        out_group(grp, rows_buf)

    fire_idx(0, idx_v0, semI0)

    def pair(t, carry):
        half(t, 2 * t, idx_v0, rows_v0, semI0,
             lambda: fire_idx(2 * t + 1, idx_v1, semI1))
        half(t, 2 * t + 1, idx_v1, rows_v1, semI1,
             lambda: fire_idx(2 * t + 2, idx_v0, semI0))
        return carry

    lax.fori_loop(0, NG // 2, pair, 0)
    wait_idx(idx_v0, semI0)
    drain_outs(rows_v0)
    hs = gather_group(idx_v0, rows_v0)
    for h in hs:
        h.wait()
    drain_outs(rows_v1)
    out_group(NG - 1, rows_v0)
    drain_outs(rows_v0)


_sc_gather = functools.partial(
    pl.kernel,
    out_type=jax.ShapeDtypeStruct((KP1, B, D), jnp.float32),
    mesh=plsc.VectorSubcoreMesh(core_axis_name="c", subcore_axis_name="s"),
    compiler_params=pltpu.CompilerParams(use_tc_tiling_on_sc=False,
                                         needs_layout_passes=False),
    scratch_types=[
        pltpu.VMEM((GROUP_ROWS,), jnp.int32),
        pltpu.VMEM((GROUP_ROWS,), jnp.int32),
        pltpu.VMEM((GROUP_ROWS, D), jnp.float32),
        pltpu.VMEM((GROUP_ROWS, D), jnp.float32),
        pltpu.SemaphoreType.DMA,
        pltpu.SemaphoreType.DMA,
        pltpu.SemaphoreType.DMA,
        pltpu.SemaphoreType.DMA,
    ],
)(_sc_gather_body)


def _sc_update_body(y_hbm, v1_hbm, v2_hbm, m1_hbm, m2_hbm,
                    nm1_hbm, nm2_hbm,
                    y_v, ids_v, yvals_v, slab_v, vrows_v, mrows_v, urows_v,
                    semC):
    cid = lax.axis_index("c")
    sid = lax.axis_index("s")
    wid = sid * 2 + cid
    iota16 = lax.iota(jnp.int32, 16)
    lo = wid * SLAB

    pltpu.sync_copy(y_hbm, y_v)

    def memset_body(j, c):
        z = jnp.zeros((16,), jnp.int32)
        ids_v[pl.ds(j * 16, 16)] = z
        yvals_v[pl.ds(j * 16, 16)] = z
        return c

    lax.fori_loop(0, (B + 128) // 16, memset_body, 0)

    def scan_body(j, count):
        yv = y_v[pl.ds(j * 16, 16)]
        lov = jnp.full((16,), lo, jnp.int32)
        hiv = jnp.full((16,), lo + SLAB, jnp.int32)
        m = (yv >= lov) & (yv < hiv)
        mi = jnp.where(m, jnp.ones((16,), jnp.int32), jnp.zeros((16,), jnp.int32))
        pos = jnp.full((16,), count - 1, jnp.int32) + plsc.cumsum(mi)
        plsc.store_scatter(ids_v, [pos],
                           jnp.full((16,), j * 16, jnp.int32) + iota16, mask=m)
        plsc.store_scatter(yvals_v, [pos], yv, mask=m)
        return count + jnp.sum(mi)

    count = lax.fori_loop(0, NGRP_Y, scan_body, 0)
    nchunks = (count + CHUNK - 1) // CHUNK

    def do_bank(mem_hbm, v_hbm, nm_hbm):
        pltpu.sync_copy(mem_hbm.at[pl.ds(lo, SLAB)], slab_v)

        def chunk_body(c, carry):
            c0 = c * CHUNK
            pltpu.async_copy(v_hbm.at[ids_v.at[pl.ds(c0, CHUNK)]],
                             vrows_v, semC).wait()
            pltpu.async_copy(mem_hbm.at[yvals_v.at[pl.ds(c0, CHUNK)]],
                             mrows_v, semC).wait()

            def sub_body(s, c2):
                rowid = s * 16 + iota16
                comps = []
                acc = jnp.zeros((16,), jnp.float32)
                for jc in range(D):
                    jv = jnp.full((16,), jc, jnp.int32)
                    vj = plsc.load_gather(vrows_v, [rowid, jv])
                    mj = plsc.load_gather(mrows_v, [rowid, jv])
                    bj = MOM * mj + (1.0 - MOM) * vj
                    comps.append(bj)
                    acc = acc + bj * bj
                rinv = _rsqrt16(acc)
                for jc in range(D):
                    jv = jnp.full((16,), jc, jnp.int32)
                    plsc.store_scatter(urows_v, [rowid, jv], comps[jc] * rinv)
                return c2

            lax.fori_loop(0, CHUNK // 16, sub_body, 0)
            rem = jnp.minimum(CHUNK, count - c0)

            def st_body(q, c2):
                yq = yvals_v[pl.ds(c0 + q, 16)][0]
                locv = jnp.full((16,), yq - lo, jnp.int32)
                qv = jnp.full((16,), q, jnp.int32)
                row = plsc.load_gather(urows_v, [qv, iota16])
                plsc.store_scatter(slab_v, [locv, iota16], row)
                return c2

            lax.fori_loop(0, rem, st_body, 0)
            return carry

        lax.fori_loop(0, nchunks, chunk_body, 0)
        pltpu.sync_copy(slab_v, nm_hbm.at[pl.ds(lo, SLAB)])

    do_bank(m1_hbm, v1_hbm, nm1_hbm)
    do_bank(m2_hbm, v2_hbm, nm2_hbm)


_sc_update = functools.partial(
    pl.kernel,
    out_type=[
        jax.ShapeDtypeStruct((V, D), jnp.float32),
        jax.ShapeDtypeStruct((V, D), jnp.float32),
    ],
    mesh=plsc.VectorSubcoreMesh(core_axis_name="c", subcore_axis_name="s"),
    compiler_params=pltpu.CompilerParams(use_tc_tiling_on_sc=False,
                                         needs_layout_passes=False),
    scratch_types=[
        pltpu.VMEM((B,), jnp.int32),
        pltpu.VMEM((B + 128,), jnp.int32),
        pltpu.VMEM((B + 128,), jnp.int32),
        pltpu.VMEM((SLAB, D), jnp.float32),
        pltpu.VMEM((CHUNK, D), jnp.float32),
        pltpu.VMEM((CHUNK, D), jnp.float32),
        pltpu.VMEM((CHUNK, D), jnp.float32),
        pltpu.SemaphoreType.DMA,
    ],
)(_sc_update_body)


BT = 16384


def _tc_body(g_ref, v1_ref, v2_ref,
             Ws2_ref, Wt2_ref, Ws1_ref, Wt1_ref, Wsv_ref, Wtv_ref,
             cs_ref, ct_ref, bsv_ref, btv_ref,
             outt_ref, outs_ref, s1T, t1T):
    k = pl.program_id(1)

    def dotT(W, x):
        return lax.dot_general(W, x, (((1,), (0,)), ((), ())),
                               precision=lax.Precision.DEFAULT,
                               preferred_element_type=jnp.float32)

    @pl.when(k == 0)
    def _():
        s1T[...] = dotT(Ws1_ref[...], v1_ref[...]) + cs_ref[...]
        t1T[...] = dotT(Wt1_ref[...], v2_ref[...]) + ct_ref[...]

    g = g_ref[0]
    rs = jnp.maximum(s1T[...] - dotT(Ws2_ref[...], g), 0.0)
    outs_ref[0] = dotT(Wsv_ref[...], rs) + bsv_ref[...]
    rt = jnp.maximum(t1T[...] - dotT(Wt2_ref[...], g), 0.0)
    outt_ref[0] = dotT(Wtv_ref[...], rt) + btv_ref[...]


def _tc_call(gT, v1T, v2T, Ws2, Wt2, Ws1, Wt1, Wsv, Wtv, cs, ct, bsv, btv):
    wspec = pl.BlockSpec((D, D), lambda i, k: (0, 0))
    bspec = pl.BlockSpec((D, 1), lambda i, k: (0, 0))
    return pl.pallas_call(
        _tc_body,
        grid=(B // BT, KP1),
        in_specs=[
            pl.BlockSpec((1, D, BT), lambda i, k: (k, 0, i)),
            pl.BlockSpec((D, BT), lambda i, k: (0, i)),
            pl.BlockSpec((D, BT), lambda i, k: (0, i)),
            wspec, wspec, wspec, wspec, wspec, wspec,
            bspec, bspec, bspec, bspec,
        ],
        out_specs=[
            pl.BlockSpec((1, D, BT), lambda i, k: (k, 0, i)),
            pl.BlockSpec((1, D, BT), lambda i, k: (k, 0, i)),
        ],
        out_shape=[
            jax.ShapeDtypeStruct((KP1, D, B), jnp.float32),
            jax.ShapeDtypeStruct((KP1, D, B), jnp.float32),
        ],
        scratch_shapes=[
            pltpu.VMEM((D, BT), jnp.float32),
            pltpu.VMEM((D, BT), jnp.float32),
        ],
    )(gT, v1T, v2T, Ws2, Wt2, Ws1, Wt1, Wsv, Wtv, cs, ct, bsv, btv)


def kernel(v1, v2, y, idx, memory_v1, memory_v2,
           w_s_v1_W, w_s_v1_b, w_s_v2_W, w_s_v2_b, w_s_v_W, w_s_v_b,
           w_t_v1_W, w_t_v1_b, w_t_v2_W, w_t_v2_b, w_t_v_W, w_t_v_b):
    idx_t = idx.T.reshape(-1)
    g3 = _sc_gather(idx_t, memory_v2)
    nm1, nm2 = _sc_update(y, v1, v2, memory_v1, memory_v2)
    gT = g3.transpose(0, 2, 1)
    cs = (w_s_v1_b - w_s_v2_b).reshape(D, 1)
    ct = (w_t_v1_b - w_t_v2_b).reshape(D, 1)
    outtT, outsT = _tc_call(gT, v1.T, v2.T, w_s_v2_W, w_t_v2_W, w_s_v1_W,
                            w_t_v1_W, w_s_v_W, w_t_v_W, cs, ct,
                            w_s_v_b.reshape(D, 1), w_t_v_b.reshape(D, 1))
    return (outtT.transpose(0, 2, 1), outsT.transpose(0, 2, 1), nm1, nm2)
